# baseline (speedup 1.0000x reference)
import jax
import jax.numpy as jnp
from jax.experimental import pallas as pl
from jax.experimental.pallas import tpu as pltpu

_BM = 256
_NBUF = 4


def _mm_manual(a_hbm, v_ref, o_ref, bufs, sems):
    steps = a_hbm.shape[0] // _BM
    groups = steps // _NBUF

    def cp(slot, blk):
        return pltpu.make_async_copy(
            a_hbm.at[pl.ds(blk * _BM, _BM), :], bufs.at[slot], sems.at[slot]
        )

    for s in range(_NBUF):
        cp(s, s).start()

    v = v_ref[...]

    def group(g, carry):
        base = g * _NBUF
        for s in range(_NBUF):
            i = base + s
            cp(s, i).wait()
            o_ref[pl.ds(i * _BM, _BM), :] = jnp.dot(
                bufs[s], v, preferred_element_type=jnp.float32
            )

            @pl.when(i + _NBUF < steps)
            def _next():
                cp(s, i + _NBUF).start()

        return carry

    jax.lax.fori_loop(0, groups, group, 0)


def kernel(matrix, matrix_batch):
    m, k = matrix.shape
    b, _, f = matrix_batch.shape
    n = b * f
    vectors = jnp.swapaxes(matrix_batch, 0, 1).reshape(k, n)

    out = pl.pallas_call(
        _mm_manual,
        in_specs=[
            pl.BlockSpec(memory_space=pltpu.MemorySpace.HBM),
            pl.BlockSpec(memory_space=pltpu.MemorySpace.VMEM),
        ],
        out_specs=pl.BlockSpec(memory_space=pltpu.MemorySpace.VMEM),
        out_shape=jax.ShapeDtypeStruct((m, n), jnp.float32),
        scratch_shapes=[
            pltpu.VMEM((_NBUF, _BM, k), jnp.float32),
            pltpu.SemaphoreType.DMA((_NBUF,)),
        ],
    )(matrix, vectors)

    return jnp.swapaxes(out.reshape(m, b, f), 0, 1)


# speedup vs baseline: 1.0477x; 1.0477x over previous
import jax
import jax.numpy as jnp
from jax.experimental import pallas as pl
from jax.experimental.pallas import tpu as pltpu

_BM = 256


def _mm(a_ref, v_ref, o_ref):
    o_ref[...] = jnp.dot(a_ref[...], v_ref[...], preferred_element_type=jnp.float32)


def kernel(matrix, matrix_batch):
    m, k = matrix.shape
    b, _, f = matrix_batch.shape
    n = b * f
    vectors = jnp.swapaxes(matrix_batch, 0, 1).reshape(k, n)

    out = pl.pallas_call(
        _mm,
        grid=(m // _BM,),
        in_specs=[
            pl.BlockSpec((_BM, k), lambda i: (i, 0)),
            pl.BlockSpec((k, n), lambda i: (0, 0)),
        ],
        out_specs=pl.BlockSpec((_BM, n), lambda i: (i, 0)),
        out_shape=jax.ShapeDtypeStruct((m, n), jnp.float32),
        compiler_params=pltpu.CompilerParams(
            dimension_semantics=(pltpu.PARALLEL,),
            allow_input_fusion=[False, True],
        ),
    )(matrix, vectors)

    return jnp.swapaxes(out.reshape(m, b, f), 0, 1)
